# baseline (speedup 1.0000x reference)
import jax
import jax.numpy as jnp
import numpy as np
from jax import lax
from jax.experimental import pallas as pl
from jax.experimental.pallas import tpu as pltpu
from jax.experimental.pallas import tpu_sc as plsc

_VOCAB = 100000
_EMBED = 128
_BATCH = 1024
_SEQLEN = 200

_NW = 32
_ROWS_PER_W = (_BATCH * _SEQLEN) // _NW
_HALF = _SEQLEN // 2
_NCHUNK = _ROWS_PER_W // _SEQLEN
_NIDX = _ROWS_PER_W // _HALF
_PA = 104
_PB = _SEQLEN - _PA


def _positional_encoding_np():
    pos = np.arange(_SEQLEN, dtype=np.float32)[:, None]
    div = np.exp(
        np.arange(0, _EMBED, 2, dtype=np.float32) * (-np.log(10000.0) / _EMBED)
    )
    ang = pos * div[None, :]
    pe = np.zeros((_SEQLEN, _EMBED), dtype=np.float32)
    pe[:, 0::2] = np.sin(ang)
    pe[:, 1::2] = np.cos(ang)
    return pe


_PE = _positional_encoding_np()


def _sc_kernel(table_hbm, idx_hbm, pe_hbm, out_hbm,
               idx_v, pe_v, b0, b1, sha, shb,
               g0, g1, ha, hb, oa, ob):
    bufs = (b0, b1)
    gsem = (g0, g1)
    sid = lax.axis_index("s")

    nc = 2
    wid = sid * nc + lax.axis_index("c")
    row_base = wid * _ROWS_PER_W

    def issue_gather(c, b):
        pltpu.async_copy(table_hbm.at[idx_v.at[2 * c]],
                         bufs[b].at[pl.ds(0, _HALF)], gsem[b])
        pltpu.async_copy(table_hbm.at[idx_v.at[2 * c + 1]],
                         bufs[b].at[pl.ds(_HALF, _HALF)], gsem[b])

    def wait_gather(c, b):
        pltpu.make_async_copy(table_hbm.at[idx_v.at[2 * c]],
                              bufs[b].at[pl.ds(0, _HALF)], gsem[b]).wait()
        pltpu.make_async_copy(table_hbm.at[idx_v.at[2 * c + 1]],
                              bufs[b].at[pl.ds(_HALF, _HALF)], gsem[b]).wait()

    def issue_hops(b):
        pltpu.async_copy(bufs[b].at[pl.ds(0, _PA)], sha.at[sid], ha)
        pltpu.async_copy(bufs[b].at[pl.ds(_PA, _PB)], shb.at[sid], hb)

    def wait_hops(b):
        pltpu.make_async_copy(bufs[b].at[pl.ds(0, _PA)], sha.at[sid],
                              ha).wait()
        pltpu.make_async_copy(bufs[b].at[pl.ds(_PA, _PB)], shb.at[sid],
                              hb).wait()

    def issue_outs(c):
        pltpu.async_copy(
            sha.at[sid], out_hbm.at[pl.ds(row_base + c * _SEQLEN, _PA)], oa)
        pltpu.async_copy(
            shb.at[sid],
            out_hbm.at[pl.ds(row_base + c * _SEQLEN + _PA, _PB)], ob)

    def wait_outs(c):
        pltpu.make_async_copy(
            sha.at[sid], out_hbm.at[pl.ds(row_base + c * _SEQLEN, _PA)],
            oa).wait()
        pltpu.make_async_copy(
            shb.at[sid],
            out_hbm.at[pl.ds(row_base + c * _SEQLEN + _PA, _PB)], ob).wait()

    def pe_add(b):
        def body(i, _):
            r = 2 * i
            for dr in range(2):
                for k in range(_EMBED // 16):
                    plsc.addupdate(
                        bufs[b].at[r + dr, pl.ds(k * 16, 16)],
                        pe_v[r + dr, pl.ds(k * 16, 16)],
                    )
            return 0

        lax.fori_loop(0, _SEQLEN // 2, body, 0)

    pltpu.sync_copy(idx_hbm.at[pl.ds(wid * _NIDX, _NIDX)], idx_v)
    issue_gather(0, 0)
    issue_gather(1, 1)
    pltpu.sync_copy(pe_hbm, pe_v)

    wait_gather(0, 0)
    pe_add(0)
    issue_hops(0)
    wait_hops(0)
    issue_outs(0)
    issue_gather(2, 0)

    def round_body(r, _):
        for j in range(2):
            c = 1 + 2 * r + j
            b = (1 + j) % 2
            wait_gather(c, b)
            pe_add(b)
            wait_outs(c - 1)
            issue_hops(b)
            wait_hops(b)
            issue_outs(c)
            issue_gather(c + 2, b)
        return 0

    lax.fori_loop(0, (_NCHUNK - 4) // 2, round_body, 0)

    for c in range(_NCHUNK - 3, _NCHUNK):
        b = c % 2
        wait_gather(c, b)
        pe_add(b)
        wait_outs(c - 1)
        issue_hops(b)
        wait_hops(b)
        issue_outs(c)
        if c + 2 < _NCHUNK:
            issue_gather(c + 2, b)
    wait_outs(_NCHUNK - 1)


@jax.jit
def _run(sequence_flat2d, token_table, pe):
    mesh = plsc.VectorSubcoreMesh(core_axis_name="c", subcore_axis_name="s")
    return pl.kernel(
        _sc_kernel,
        mesh=mesh,
        out_type=jax.ShapeDtypeStruct((_BATCH * _SEQLEN, _EMBED), jnp.float32),
        scratch_types=[
            pltpu.VMEM((_NIDX, _HALF), jnp.int32),
            pltpu.VMEM((_SEQLEN, _EMBED), jnp.float32),
            pltpu.VMEM((_SEQLEN, _EMBED), jnp.float32),
            pltpu.VMEM((_SEQLEN, _EMBED), jnp.float32),
            pltpu.VMEM_SHARED((16, _PA, _EMBED), jnp.float32),
            pltpu.VMEM_SHARED((16, _PB, _EMBED), jnp.float32),
            pltpu.SemaphoreType.DMA,
            pltpu.SemaphoreType.DMA,
            pltpu.SemaphoreType.DMA,
            pltpu.SemaphoreType.DMA,
            pltpu.SemaphoreType.DMA,
            pltpu.SemaphoreType.DMA,
        ],
    )(token_table, sequence_flat2d, pe)


def kernel(sequence, token_table):
    idx = sequence.reshape(-1).astype(jnp.int32).reshape(-1, _HALF)
    pe = jnp.asarray(_PE)
    out = _run(idx, token_table, pe)
    return out.reshape(_BATCH, _SEQLEN, _EMBED)


# speedup vs baseline: 1.1339x; 1.1339x over previous
import jax
import jax.numpy as jnp
import numpy as np
from jax import lax
from jax.experimental import pallas as pl
from jax.experimental.pallas import tpu as pltpu
from jax.experimental.pallas import tpu_sc as plsc

_VOCAB = 100000
_EMBED = 128
_BATCH = 1024
_SEQLEN = 200

_NW = 32
_ROWS_PER_W = (_BATCH * _SEQLEN) // _NW
_HALF = _SEQLEN // 2
_NCHUNK = _ROWS_PER_W // _SEQLEN
_NIDX = _ROWS_PER_W // _HALF
_PA = 96
_PB = _SEQLEN - _PA


def _positional_encoding_np():
    pos = np.arange(_SEQLEN, dtype=np.float32)[:, None]
    div = np.exp(
        np.arange(0, _EMBED, 2, dtype=np.float32) * (-np.log(10000.0) / _EMBED)
    )
    ang = pos * div[None, :]
    pe = np.zeros((_SEQLEN, _EMBED), dtype=np.float32)
    pe[:, 0::2] = np.sin(ang)
    pe[:, 1::2] = np.cos(ang)
    return pe


_PE = _positional_encoding_np()


def _sc_kernel(table_hbm, idx_hbm, pe_hbm, out_hbm,
               idx_v, pe_v, b0, b1, sha, shb,
               ga0, gb0, ga1, gb1, ha, hb, oa, ob):
    bufs = (b0, b1)
    gsa = (ga0, ga1)
    gsb = (gb0, gb1)
    sid = lax.axis_index("s")

    nc = 2
    wid = sid * nc + lax.axis_index("c")
    row_base = wid * _ROWS_PER_W

    def issue_gather(c, b):
        pltpu.async_copy(table_hbm.at[idx_v.at[2 * c]],
                         bufs[b].at[pl.ds(0, _HALF)], gsa[b])
        pltpu.async_copy(table_hbm.at[idx_v.at[2 * c + 1]],
                         bufs[b].at[pl.ds(_HALF, _HALF)], gsb[b])

    def wait_stream_a(c, b):
        pltpu.make_async_copy(table_hbm.at[idx_v.at[2 * c]],
                              bufs[b].at[pl.ds(0, _HALF)], gsa[b]).wait()

    def wait_stream_b(c, b):
        pltpu.make_async_copy(table_hbm.at[idx_v.at[2 * c + 1]],
                              bufs[b].at[pl.ds(_HALF, _HALF)], gsb[b]).wait()

    def hop_a(b):
        pltpu.async_copy(bufs[b].at[pl.ds(0, _PA)], sha.at[sid], ha)

    def hop_b(b):
        pltpu.async_copy(bufs[b].at[pl.ds(_PA, _PB)], shb.at[sid], hb)

    def wait_hop_a(b):
        pltpu.make_async_copy(bufs[b].at[pl.ds(0, _PA)], sha.at[sid],
                              ha).wait()

    def wait_hop_b(b):
        pltpu.make_async_copy(bufs[b].at[pl.ds(_PA, _PB)], shb.at[sid],
                              hb).wait()

    def out_a(c):
        pltpu.async_copy(
            sha.at[sid], out_hbm.at[pl.ds(row_base + c * _SEQLEN, _PA)], oa)

    def out_b(c):
        pltpu.async_copy(
            shb.at[sid],
            out_hbm.at[pl.ds(row_base + c * _SEQLEN + _PA, _PB)], ob)

    def wait_out_a(c):
        pltpu.make_async_copy(
            sha.at[sid], out_hbm.at[pl.ds(row_base + c * _SEQLEN, _PA)],
            oa).wait()

    def wait_out_b(c):
        pltpu.make_async_copy(
            shb.at[sid],
            out_hbm.at[pl.ds(row_base + c * _SEQLEN + _PA, _PB)], ob).wait()

    def pe_add(b, lo, hi):
        def body(i, _):
            r = lo + 2 * i
            for dr in range(2):
                for k in range(_EMBED // 16):
                    plsc.addupdate(
                        bufs[b].at[r + dr, pl.ds(k * 16, 16)],
                        pe_v[r + dr, pl.ds(k * 16, 16)],
                    )
            return 0

        lax.fori_loop(0, (hi - lo) // 2, body, 0)

    def process(c, b, first, last):
        wait_stream_a(c, b)
        pe_add(b, 0, _PA)
        if not first:
            wait_out_a(c - 1)
        hop_a(b)
        wait_stream_b(c, b)
        pe_add(b, _PA, _SEQLEN)
        if not first:
            wait_out_b(c - 1)
        hop_b(b)
        wait_hop_a(b)
        out_a(c)
        wait_hop_b(b)
        out_b(c)
        if not last:
            issue_gather(c + 2, b)

    pltpu.sync_copy(idx_hbm.at[pl.ds(wid * _NIDX, _NIDX)], idx_v)
    issue_gather(0, 0)
    issue_gather(1, 1)
    pltpu.sync_copy(pe_hbm, pe_v)

    process(0, 0, True, False)

    def round_body(r, _):
        for j in range(2):
            c = 1 + 2 * r + j
            process(c, (1 + j) % 2, False, False)
        return 0

    lax.fori_loop(0, (_NCHUNK - 4) // 2, round_body, 0)

    for c in range(_NCHUNK - 3, _NCHUNK):
        process(c, c % 2, False, c + 2 >= _NCHUNK)
    wait_out_a(_NCHUNK - 1)
    wait_out_b(_NCHUNK - 1)


@jax.jit
def _run(sequence_flat2d, token_table, pe):
    mesh = plsc.VectorSubcoreMesh(core_axis_name="c", subcore_axis_name="s")
    return pl.kernel(
        _sc_kernel,
        mesh=mesh,
        out_type=jax.ShapeDtypeStruct((_BATCH * _SEQLEN, _EMBED), jnp.float32),
        scratch_types=[
            pltpu.VMEM((_NIDX, _HALF), jnp.int32),
            pltpu.VMEM((_SEQLEN, _EMBED), jnp.float32),
            pltpu.VMEM((_SEQLEN, _EMBED), jnp.float32),
            pltpu.VMEM((_SEQLEN, _EMBED), jnp.float32),
            pltpu.VMEM_SHARED((16, _PA, _EMBED), jnp.float32),
            pltpu.VMEM_SHARED((16, _PB, _EMBED), jnp.float32),
            pltpu.SemaphoreType.DMA,
            pltpu.SemaphoreType.DMA,
            pltpu.SemaphoreType.DMA,
            pltpu.SemaphoreType.DMA,
            pltpu.SemaphoreType.DMA,
            pltpu.SemaphoreType.DMA,
            pltpu.SemaphoreType.DMA,
            pltpu.SemaphoreType.DMA,
        ],
    )(token_table, sequence_flat2d, pe)


def kernel(sequence, token_table):
    idx = sequence.reshape(-1).astype(jnp.int32).reshape(-1, _HALF)
    pe = jnp.asarray(_PE)
    out = _run(idx, token_table, pe)
    return out.reshape(_BATCH, _SEQLEN, _EMBED)
